# Initial kernel scaffold; baseline (speedup 1.0000x reference)
import jax
import jax.numpy as jnp
from jax.experimental import pallas as pl


def kernel(bank, weight_alloc):
    raise NotImplementedError("write your pallas kernel here")


# speedup vs baseline: 233.3997x; 233.3997x over previous
import functools

import jax
import jax.numpy as jnp
from jax import lax
from jax.experimental import pallas as pl
from jax.experimental.pallas import tpu as pltpu
from jax.experimental.pallas import tpu_sc as plsc

_FULL_ROWS = 100_000
_FULL_COLS = 128
_N_ALLOC = _FULL_ROWS * _FULL_COLS
_NUM_CORES = 2
_NUM_SUBCORES = 16
_NW = _NUM_CORES * _NUM_SUBCORES
_PER_W = _N_ALLOC // _NW
_CHUNK = 25_000
_NCHUNK = _PER_W // _CHUNK


def _make_gather():
    mesh = plsc.VectorSubcoreMesh(
        core_axis_name="c", subcore_axis_name="s")

    @functools.partial(
        pl.kernel,
        mesh=mesh,
        out_type=jax.ShapeDtypeStruct((_N_ALLOC,), jnp.float32),
        scratch_types=[
            pltpu.VMEM((_CHUNK,), jnp.int32),
            pltpu.VMEM((_CHUNK,), jnp.float32),
            pltpu.SemaphoreType.DMA,
        ],
    )
    def gather_kernel(bank_hbm, wa_hbm, out_hbm, idx_v, rows_v, sem):
        wid = lax.axis_index("c") * _NUM_SUBCORES + lax.axis_index("s")
        base = wid * _PER_W

        def body(i, carry):
            off = base + i * _CHUNK
            pltpu.sync_copy(wa_hbm.at[pl.ds(off, _CHUNK)], idx_v)
            pltpu.async_copy(bank_hbm.at[idx_v], rows_v, sem).wait()
            pltpu.sync_copy(rows_v, out_hbm.at[pl.ds(off, _CHUNK)])
            return carry

        lax.fori_loop(0, _NCHUNK, body, 0)

    return gather_kernel


_gather = _make_gather()


@jax.jit
def kernel(bank, weight_alloc):
    wa = weight_alloc.reshape(-1).astype(jnp.int32)
    out = _gather(bank, wa)
    return out.reshape(_FULL_ROWS, _FULL_COLS)


# speedup vs baseline: 836.2845x; 3.5831x over previous
import functools

import jax
import jax.numpy as jnp
from jax import lax
from jax.experimental import pallas as pl
from jax.experimental.pallas import tpu as pltpu
from jax.experimental.pallas import tpu_sc as plsc

_FULL_ROWS = 100_000
_FULL_COLS = 128
_N_ALLOC = _FULL_ROWS * _FULL_COLS
_NUM_BANK = 1_280_000
_NUM_CORES = 2
_NUM_SUBCORES = 16
_NW = _NUM_CORES * _NUM_SUBCORES
_PER_W = _N_ALLOC // _NW
_CHUNK = 10_000
_NCHUNK = _PER_W // _CHUNK
_NPAIR = _NCHUNK // 2
_BANK_SLICE = _NUM_BANK // _NUM_SUBCORES


def _make_gather():
    mesh = plsc.VectorSubcoreMesh(
        core_axis_name="c", subcore_axis_name="s")

    @functools.partial(
        pl.kernel,
        mesh=mesh,
        out_type=jax.ShapeDtypeStruct((_N_ALLOC,), jnp.float32),
        scratch_types=[
            pltpu.VMEM_SHARED((_NUM_BANK,), jnp.float32),
            pltpu.VMEM((_CHUNK,), jnp.int32),
            pltpu.VMEM((_CHUNK,), jnp.int32),
            pltpu.VMEM((_CHUNK,), jnp.float32),
            pltpu.VMEM((_CHUNK,), jnp.float32),
            pltpu.SemaphoreType.DMA,
            pltpu.SemaphoreType.DMA,
            pltpu.SemaphoreType.DMA,
            pltpu.SemaphoreType.DMA,
            pltpu.SemaphoreType.DMA,
        ],
    )
    def gather_kernel(bank_hbm, wa_hbm, out_hbm, bank_sh,
                      idx_a, idx_b, rows_a, rows_b,
                      sem_ia, sem_ib, sem_oa, sem_ob, sem_g):
        sid = lax.axis_index("s")
        wid = lax.axis_index("c") * _NUM_SUBCORES + sid
        base = wid * _PER_W

        boff = sid * _BANK_SLICE
        pltpu.sync_copy(bank_hbm.at[pl.ds(boff, _BANK_SLICE)],
                        bank_sh.at[pl.ds(boff, _BANK_SLICE)])
        plsc.subcore_barrier()

        pltpu.async_copy(wa_hbm.at[pl.ds(base, _CHUNK)], idx_a, sem_ia)
        pltpu.async_copy(wa_hbm.at[pl.ds(base + _CHUNK, _CHUNK)], idx_b,
                         sem_ib)

        bufs = ((idx_a, rows_a, sem_ia, sem_oa),
                (idx_b, rows_b, sem_ib, sem_ob))

        def body(i, carry):
            for b, (idx_v, rows_v, sem_i, sem_o) in enumerate(bufs):
                off = base + (2 * i + b) * _CHUNK

                @pl.when(i > 0)
                def _():
                    pltpu.make_async_copy(
                        rows_v, out_hbm.at[pl.ds(base, _CHUNK)], sem_o
                    ).wait()

                pltpu.make_async_copy(
                    wa_hbm.at[pl.ds(off, _CHUNK)], idx_v, sem_i).wait()

                pltpu.async_copy(bank_sh.at[idx_v], rows_v, sem_g).wait()

                pltpu.async_copy(rows_v, out_hbm.at[pl.ds(off, _CHUNK)],
                                 sem_o)

                @pl.when(i < _NPAIR - 1)
                def _():
                    pltpu.async_copy(
                        wa_hbm.at[pl.ds(off + 2 * _CHUNK, _CHUNK)],
                        idx_v, sem_i)
            return carry

        lax.fori_loop(0, _NPAIR, body, 0)

        pltpu.make_async_copy(
            rows_a, out_hbm.at[pl.ds(base, _CHUNK)], sem_oa).wait()
        pltpu.make_async_copy(
            rows_b, out_hbm.at[pl.ds(base, _CHUNK)], sem_ob).wait()

    return gather_kernel


_gather = _make_gather()


@jax.jit
def kernel(bank, weight_alloc):
    wa = weight_alloc.reshape(-1).astype(jnp.int32)
    out = _gather(bank, wa)
    return out.reshape(_FULL_ROWS, _FULL_COLS)


# speedup vs baseline: 872.3596x; 1.0431x over previous
import functools

import jax
import jax.numpy as jnp
from jax import lax
from jax.experimental import pallas as pl
from jax.experimental.pallas import tpu as pltpu
from jax.experimental.pallas import tpu_sc as plsc

_FULL_ROWS = 100_000
_FULL_COLS = 128
_N_ALLOC = _FULL_ROWS * _FULL_COLS
_NUM_BANK = 1_280_000
_NUM_CORES = 2
_NUM_SUBCORES = 16
_NW = _NUM_CORES * _NUM_SUBCORES
_PER_W = _N_ALLOC // _NW
_CHUNK = 5_000
_NCHUNK = _PER_W // _CHUNK
_NBUF = 4
_NITER = _NCHUNK // _NBUF
_BANK_SLICE = _NUM_BANK // _NUM_SUBCORES


def _make_gather():
    mesh = plsc.VectorSubcoreMesh(
        core_axis_name="c", subcore_axis_name="s")

    @functools.partial(
        pl.kernel,
        mesh=mesh,
        out_type=jax.ShapeDtypeStruct((_N_ALLOC,), jnp.float32),
        scratch_types=(
            [pltpu.VMEM_SHARED((_NUM_BANK,), jnp.float32)]
            + [pltpu.VMEM((_CHUNK,), jnp.int32) for _ in range(_NBUF)]
            + [pltpu.VMEM((_CHUNK,), jnp.float32) for _ in range(_NBUF)]
            + [pltpu.SemaphoreType.DMA for _ in range(3 * _NBUF)]
        ),
    )
    def gather_kernel(bank_hbm, wa_hbm, out_hbm, bank_sh, *bufs):
        idx_v = bufs[0:_NBUF]
        rows_v = bufs[_NBUF:2 * _NBUF]
        sem_i = bufs[2 * _NBUF:3 * _NBUF]
        sem_o = bufs[3 * _NBUF:4 * _NBUF]
        sem_g = bufs[4 * _NBUF:5 * _NBUF]

        sid = lax.axis_index("s")
        wid = lax.axis_index("c") * _NUM_SUBCORES + sid
        base = wid * _PER_W

        def idx_start(k, j):
            pltpu.async_copy(
                wa_hbm.at[pl.ds(base + k * _CHUNK, _CHUNK)],
                idx_v[j], sem_i[j])

        def idx_wait(j):
            pltpu.make_async_copy(
                wa_hbm.at[pl.ds(base, _CHUNK)], idx_v[j], sem_i[j]).wait()

        def gather_start(j):
            pltpu.async_copy(bank_sh.at[idx_v[j]], rows_v[j], sem_g[j])

        def gather_wait(j):
            pltpu.make_async_copy(
                bank_sh.at[idx_v[j]], rows_v[j], sem_g[j]).wait()

        def out_start(k, j):
            pltpu.async_copy(
                rows_v[j], out_hbm.at[pl.ds(base + k * _CHUNK, _CHUNK)],
                sem_o[j])

        def out_wait(j):
            pltpu.make_async_copy(
                rows_v[j], out_hbm.at[pl.ds(base, _CHUNK)], sem_o[j]).wait()

        idx_start(0, 0)
        idx_start(1, 1)

        boff = sid * _BANK_SLICE
        pltpu.sync_copy(bank_hbm.at[pl.ds(boff, _BANK_SLICE)],
                        bank_sh.at[pl.ds(boff, _BANK_SLICE)])
        plsc.subcore_barrier()

        def body(i, carry):
            for j in range(_NBUF):
                k = i * _NBUF + j
                jp2 = (j + 2) % _NBUF

                idx_wait(j)

                @pl.when(k >= _NBUF)
                def _():
                    out_wait(j)

                gather_start(j)

                @pl.when(k >= 2)
                def _():
                    gather_wait(jp2)
                    out_start_k = k - 2
                    pltpu.async_copy(
                        rows_v[jp2],
                        out_hbm.at[pl.ds(base + out_start_k * _CHUNK,
                                         _CHUNK)],
                        sem_o[jp2])

                @pl.when(k + 2 < _NCHUNK)
                def _():
                    idx_start(k + 2, jp2)
            return carry

        lax.fori_loop(0, _NITER, body, 0)

        gather_wait(2)
        out_start(_NCHUNK - 2, 2)
        gather_wait(3)
        out_start(_NCHUNK - 1, 3)
        for j in range(_NBUF):
            out_wait(j)

    return gather_kernel


_gather = _make_gather()


@jax.jit
def kernel(bank, weight_alloc):
    wa = weight_alloc.reshape(-1).astype(jnp.int32)
    out = _gather(bank, wa)
    return out.reshape(_FULL_ROWS, _FULL_COLS)
